# baseline (speedup 1.0000x reference)
import functools

import jax
import jax.numpy as jnp
from jax import lax
from jax.experimental import pallas as pl
from jax.experimental.pallas import tpu as pltpu

_D = 64
_NCLS = 10
_C = 1024
_KAPPA_N = 5.0
_S0 = 0.001
_B = 256
_CB = 128
_NBLK = _C // _CB


def _sigma_kernel(s_ref, nmax_ref, sig_ref):
    si = lax.broadcasted_iota(jnp.int32, (_CB * _D, _D), 0)
    sj = lax.broadcasted_iota(jnp.int32, (_CB * _D, _D), 1)
    eye = jnp.where(jnp.bitwise_and(si, _D - 1) == sj, _S0, 0.0)
    sig_ref[...] = s_ref[...] / nmax_ref[...] + eye


def _dist_kernel(sb_ref, piv_ref, data_ref, mu_ref, n_ref, gam_ref):
    li = lax.broadcasted_iota(jnp.int32, (_CB, _D), 1)
    perm = li
    for i in range(_D):
        pj = piv_ref[:, i:i + 1]
        y = jnp.sum(jnp.where(li == pj, perm, 0), axis=1, keepdims=True)
        x = perm[:, i:i + 1]
        perm = jnp.where(li == i, y, perm)
        perm = jnp.where((li == pj) & (li != i), x, perm)

    data = data_ref[...]
    sub = lax.broadcasted_iota(jnp.int32, (_D, _D), 0)
    cols = []
    for ci in range(_CB):
        row = lax.slice(perm, (ci, 0), (ci + 1, _D))
        p_t = jnp.where(sub == row, 1.0, 0.0).astype(jnp.bfloat16)
        sb_c = sb_ref[ci * _D:(ci + 1) * _D, :]
        sbp = lax.dot_general(sb_c, p_t, (((1,), (1,)), ((), ())),
                              preferred_element_type=jnp.float32
                              ).astype(jnp.bfloat16)
        diff = data - mu_ref[ci:ci + 1, :]
        tmp = jnp.dot(diff.astype(jnp.bfloat16), sbp,
                      preferred_element_type=jnp.float32)
        cols.append(jnp.sum(tmp * diff, axis=1, keepdims=True))
    d2 = jnp.concatenate(cols, axis=1)
    match = jnp.where(n_ref[0] >= _KAPPA_N, 1.0, 0.0)
    gam_ref[...] = jnp.exp(-0.5 * d2) * match


def _score_kernel(gam_ref, lab_ref, sc_ref, pred_ref, cl_ref):
    g = gam_ref[...]
    den = jnp.sum(g, axis=1, keepdims=True) + 1e-12
    gn = (g / den).astype(jnp.bfloat16)
    sc = jnp.dot(gn, lab_ref[...], preferred_element_type=jnp.float32)
    sc_ref[...] = sc
    li = lax.broadcasted_iota(jnp.int32, (_B, 16), 1)
    scm = jnp.where(li < _NCLS, sc, -jnp.inf)
    m = jnp.max(scm, axis=1, keepdims=True)
    pred_ref[...] = jnp.min(jnp.where(scm == m, li, 2 ** 30),
                            axis=1, keepdims=True)
    ci = lax.broadcasted_iota(jnp.int32, (_B, _C), 1)
    gm = jnp.max(g, axis=1, keepdims=True)
    cl_ref[...] = jnp.min(jnp.where(g == gm, ci, 2 ** 30),
                          axis=1, keepdims=True)


@functools.partial(jax.jit, static_argnames=("interpret",))
def _run(data, mu, S, n, cluster_labels, interpret=False):
    f32 = jnp.float32
    n_c = n[:_C].astype(f32)
    nmax_rep = jnp.repeat(jnp.maximum(n_c, 1.0), _D)[:, None]
    s2 = S[:_C].astype(f32).reshape(_C * _D, _D)

    sigma2 = pl.pallas_call(
        _sigma_kernel,
        grid=(_NBLK,),
        in_specs=[
            pl.BlockSpec((_CB * _D, _D), lambda k: (k, 0)),
            pl.BlockSpec((_CB * _D, 1), lambda k: (k, 0)),
        ],
        out_specs=pl.BlockSpec((_CB * _D, _D), lambda k: (k, 0)),
        out_shape=jax.ShapeDtypeStruct((_C * _D, _D), f32),
        interpret=interpret,
    )(s2, nmax_rep)
    Sigma = sigma2.reshape(_C, _D, _D)

    lu, pivots, _ = lax.linalg.lu(Sigma)
    eye_b = jnp.broadcast_to(jnp.eye(_D, dtype=f32), (_C, _D, _D))
    x = lax.linalg.triangular_solve(lu, eye_b, left_side=True, lower=True,
                                    unit_diagonal=True)
    x = lax.linalg.triangular_solve(lu, x, left_side=True, lower=False)
    sb = x.astype(jnp.bfloat16).reshape(_C * _D, _D)

    n3 = n_c.reshape(_NBLK, 1, _CB)

    gam = pl.pallas_call(
        _dist_kernel,
        grid=(_NBLK,),
        in_specs=[
            pl.BlockSpec((_CB * _D, _D), lambda k: (k, 0)),
            pl.BlockSpec((_CB, _D), lambda k: (k, 0)),
            pl.BlockSpec((_B, _D), lambda k: (0, 0)),
            pl.BlockSpec((_CB, _D), lambda k: (k, 0)),
            pl.BlockSpec((1, 1, _CB), lambda k: (k, 0, 0)),
        ],
        out_specs=pl.BlockSpec((_B, _CB), lambda k: (0, k)),
        out_shape=jax.ShapeDtypeStruct((_B, _C), f32),
        interpret=interpret,
    )(sb, pivots, data.astype(f32), mu[:_C].astype(f32), n3)

    lab = cluster_labels[:_C].astype(f32)
    labx = jnp.concatenate([lab, jnp.zeros((_C, 6), f32)],
                           axis=1).astype(jnp.bfloat16)

    sc16, pred, clusters = pl.pallas_call(
        _score_kernel,
        grid=(1,),
        in_specs=[
            pl.BlockSpec((_B, _C), lambda k: (0, 0)),
            pl.BlockSpec((_C, 16), lambda k: (0, 0)),
        ],
        out_specs=[
            pl.BlockSpec((_B, 16), lambda k: (0, 0)),
            pl.BlockSpec((_B, 1), lambda k: (0, 0)),
            pl.BlockSpec((_B, 1), lambda k: (0, 0)),
        ],
        out_shape=[
            jax.ShapeDtypeStruct((_B, 16), f32),
            jax.ShapeDtypeStruct((_B, 1), jnp.int32),
            jax.ShapeDtypeStruct((_B, 1), jnp.int32),
        ],
        interpret=interpret,
    )(gam, labx)

    return sc16[:, :_NCLS], pred.reshape(_B), clusters.reshape(_B)


def kernel(data, labels, mu, S, n, cluster_labels):
    del labels
    return _run(data, mu, S, n, cluster_labels)


# speedup vs baseline: 1.0106x; 1.0106x over previous
import functools

import jax
import jax.numpy as jnp
from jax import lax
from jax.experimental import pallas as pl
from jax.experimental.pallas import tpu as pltpu

_D = 64
_NCLS = 10
_C = 1024
_KAPPA_N = 5.0
_S0 = 0.001
_B = 256
_CB = 128
_NBLK = _C // _CB


def _sigma_kernel(s_ref, nmax_ref, sig_ref):
    si = lax.broadcasted_iota(jnp.int32, (_CB * _D, _D), 0)
    sj = lax.broadcasted_iota(jnp.int32, (_CB * _D, _D), 1)
    eye = jnp.where(jnp.bitwise_and(si, _D - 1) == sj, _S0, 0.0)
    sig_ref[...] = s_ref[...] / nmax_ref[...] + eye


_G = 4
_GB = _G * _D


def _dist_kernel(sb_ref, data_ref, mu4_ref, n_ref, gam_ref):
    data = data_ref[...]
    data4 = jnp.concatenate([data] * _G, axis=1)
    ri = lax.broadcasted_iota(jnp.int32, (_GB, _GB), 0) // _D
    cj = lax.broadcasted_iota(jnp.int32, (_GB, _GB), 1) // _D
    blkmask = ri == cj
    lj = lax.broadcasted_iota(jnp.int32, (_B, _GB), 1) // _D
    cols = []
    for g in range(_CB // _G):
        sbg = sb_ref[g * _GB:(g + 1) * _GB, :]
        rhs = jnp.where(blkmask,
                        jnp.concatenate([sbg] * _G, axis=1),
                        jnp.bfloat16(0.0))
        diff4 = data4 - mu4_ref[g:g + 1, :]
        tmp4 = jnp.dot(diff4.astype(jnp.bfloat16), rhs,
                       preferred_element_type=jnp.float32)
        p4 = tmp4 * diff4
        for b in range(_G):
            cols.append(jnp.sum(jnp.where(lj == b, p4, 0.0),
                                axis=1, keepdims=True))
    d2 = jnp.concatenate(cols, axis=1)
    match = jnp.where(n_ref[0] >= _KAPPA_N, 1.0, 0.0)
    gam_ref[...] = jnp.exp(-0.5 * d2) * match


def _score_kernel(gam_ref, lab_ref, sc_ref, pred_ref, cl_ref):
    g = gam_ref[...]
    den = jnp.sum(g, axis=1, keepdims=True) + 1e-12
    gn = (g / den).astype(jnp.bfloat16)
    sc = jnp.dot(gn, lab_ref[...], preferred_element_type=jnp.float32)
    sc_ref[...] = sc
    li = lax.broadcasted_iota(jnp.int32, (_B, 16), 1)
    scm = jnp.where(li < _NCLS, sc, -jnp.inf)
    m = jnp.max(scm, axis=1, keepdims=True)
    pred_ref[...] = jnp.min(jnp.where(scm == m, li, 2 ** 30),
                            axis=1, keepdims=True)
    ci = lax.broadcasted_iota(jnp.int32, (_B, _C), 1)
    gm = jnp.max(g, axis=1, keepdims=True)
    cl_ref[...] = jnp.min(jnp.where(g == gm, ci, 2 ** 30),
                          axis=1, keepdims=True)


@functools.partial(jax.jit, static_argnames=("interpret",))
def _run(data, mu, S, n, cluster_labels, interpret=False):
    f32 = jnp.float32
    n_c = n[:_C].astype(f32)
    nmax_rep = jnp.repeat(jnp.maximum(n_c, 1.0), _D)[:, None]
    s2 = S[:_C].astype(f32).reshape(_C * _D, _D)

    sigma2 = pl.pallas_call(
        _sigma_kernel,
        grid=(_NBLK,),
        in_specs=[
            pl.BlockSpec((_CB * _D, _D), lambda k: (k, 0)),
            pl.BlockSpec((_CB * _D, 1), lambda k: (k, 0)),
        ],
        out_specs=pl.BlockSpec((_CB * _D, _D), lambda k: (k, 0)),
        out_shape=jax.ShapeDtypeStruct((_C * _D, _D), f32),
        interpret=interpret,
    )(s2, nmax_rep)
    Sigma = sigma2.reshape(_C, _D, _D)

    sb = jnp.linalg.inv(Sigma).astype(jnp.bfloat16).reshape(_C * _D, _D)

    n3 = n_c.reshape(_NBLK, 1, _CB)
    mu4 = mu[:_C].astype(f32).reshape(_C // _G, _GB)

    gam = pl.pallas_call(
        _dist_kernel,
        grid=(_NBLK,),
        in_specs=[
            pl.BlockSpec((_CB * _D, _D), lambda k: (k, 0)),
            pl.BlockSpec((_B, _D), lambda k: (0, 0)),
            pl.BlockSpec((_CB // _G, _GB), lambda k: (k, 0)),
            pl.BlockSpec((1, 1, _CB), lambda k: (k, 0, 0)),
        ],
        out_specs=pl.BlockSpec((_B, _CB), lambda k: (0, k)),
        out_shape=jax.ShapeDtypeStruct((_B, _C), f32),
        interpret=interpret,
    )(sb, data.astype(f32), mu4, n3)

    lab = cluster_labels[:_C].astype(f32)
    labx = jnp.concatenate([lab, jnp.zeros((_C, 6), f32)],
                           axis=1).astype(jnp.bfloat16)

    sc16, pred, clusters = pl.pallas_call(
        _score_kernel,
        grid=(1,),
        in_specs=[
            pl.BlockSpec((_B, _C), lambda k: (0, 0)),
            pl.BlockSpec((_C, 16), lambda k: (0, 0)),
        ],
        out_specs=[
            pl.BlockSpec((_B, 16), lambda k: (0, 0)),
            pl.BlockSpec((_B, 1), lambda k: (0, 0)),
            pl.BlockSpec((_B, 1), lambda k: (0, 0)),
        ],
        out_shape=[
            jax.ShapeDtypeStruct((_B, 16), f32),
            jax.ShapeDtypeStruct((_B, 1), jnp.int32),
            jax.ShapeDtypeStruct((_B, 1), jnp.int32),
        ],
        interpret=interpret,
    )(gam, labx)

    return sc16[:, :_NCLS], pred.reshape(_B), clusters.reshape(_B)


def kernel(data, labels, mu, S, n, cluster_labels):
    del labels
    return _run(data, mu, S, n, cluster_labels)


# speedup vs baseline: 1.0126x; 1.0021x over previous
import functools

import jax
import jax.numpy as jnp
from jax import lax
from jax.experimental import pallas as pl
from jax.experimental.pallas import tpu as pltpu

_D = 64
_NCLS = 10
_C = 1024
_KAPPA_N = 5.0
_S0 = 0.001
_B = 256
_CB = 128
_NBLK = _C // _CB


def _sigma_kernel(s_ref, nmax_ref, sig_ref):
    si = lax.broadcasted_iota(jnp.int32, (_CB * _D, _D), 0)
    sj = lax.broadcasted_iota(jnp.int32, (_CB * _D, _D), 1)
    eye = jnp.where(jnp.bitwise_and(si, _D - 1) == sj, _S0, 0.0)
    sig_ref[...] = s_ref[...] / nmax_ref[...] + eye


_G = 4
_GB = _G * _D


def _dist_kernel(sb_ref, data_ref, mu4_ref, n_ref, gam_ref):
    data = data_ref[...]
    data4 = jnp.concatenate([data] * _G, axis=1)
    ri = lax.broadcasted_iota(jnp.int32, (_GB, _GB), 0) // _D
    cj = lax.broadcasted_iota(jnp.int32, (_GB, _GB), 1) // _D
    blkmask = ri == cj
    lj = lax.broadcasted_iota(jnp.int32, (_B, _GB), 1) // _D
    cols = []
    for g in range(_CB // _G):
        sbg = sb_ref[g * _GB:(g + 1) * _GB, :]
        rhs = jnp.where(blkmask,
                        jnp.concatenate([sbg] * _G, axis=1),
                        jnp.bfloat16(0.0))
        diff4 = data4 - mu4_ref[g:g + 1, :]
        tmp4 = jnp.dot(diff4.astype(jnp.bfloat16), rhs,
                       preferred_element_type=jnp.float32)
        p4 = tmp4 * diff4
        for b in range(_G):
            cols.append(jnp.sum(jnp.where(lj == b, p4, 0.0),
                                axis=1, keepdims=True))
    d2 = jnp.concatenate(cols, axis=1)
    match = jnp.where(n_ref[0] >= _KAPPA_N, 1.0, 0.0)
    gam_ref[...] = jnp.exp(-0.5 * d2) * match


def _score_kernel(gam_ref, lab_ref, sc_ref, pred_ref, cl_ref):
    g = gam_ref[...]
    den = jnp.sum(g, axis=1, keepdims=True) + 1e-12
    gn = (g / den).astype(jnp.bfloat16)
    sc = jnp.dot(gn, lab_ref[...], preferred_element_type=jnp.float32)
    sc_ref[...] = sc
    li = lax.broadcasted_iota(jnp.int32, (_B, 16), 1)
    scm = jnp.where(li < _NCLS, sc, -jnp.inf)
    m = jnp.max(scm, axis=1, keepdims=True)
    pred_ref[...] = jnp.min(jnp.where(scm == m, li, 2 ** 30),
                            axis=1, keepdims=True)
    ci = lax.broadcasted_iota(jnp.int32, (_B, _C), 1)
    gm = jnp.max(g, axis=1, keepdims=True)
    cl_ref[...] = jnp.min(jnp.where(g == gm, ci, 2 ** 30),
                          axis=1, keepdims=True)


@functools.partial(jax.jit, static_argnames=("interpret",))
def _run(data, mu, S, n, cluster_labels, interpret=False):
    f32 = jnp.float32
    n_c = n[:_C].astype(f32)
    Sigma = (S[:_C].astype(f32) / jnp.maximum(n_c, 1.0)[:, None, None]
             + _S0 * jnp.eye(_D, dtype=f32)[None])

    sb = jnp.linalg.inv(Sigma).astype(jnp.bfloat16).reshape(_C * _D, _D)

    n3 = n_c.reshape(_NBLK, 1, _CB)
    mu4 = mu[:_C].astype(f32).reshape(_C // _G, _GB)

    gam = pl.pallas_call(
        _dist_kernel,
        grid=(_NBLK,),
        in_specs=[
            pl.BlockSpec((_CB * _D, _D), lambda k: (k, 0)),
            pl.BlockSpec((_B, _D), lambda k: (0, 0)),
            pl.BlockSpec((_CB // _G, _GB), lambda k: (k, 0)),
            pl.BlockSpec((1, 1, _CB), lambda k: (k, 0, 0)),
        ],
        out_specs=pl.BlockSpec((_B, _CB), lambda k: (0, k)),
        out_shape=jax.ShapeDtypeStruct((_B, _C), f32),
        interpret=interpret,
    )(sb, data.astype(f32), mu4, n3)

    lab = cluster_labels[:_C].astype(f32)
    labx = jnp.concatenate([lab, jnp.zeros((_C, 6), f32)],
                           axis=1).astype(jnp.bfloat16)

    sc16, pred, clusters = pl.pallas_call(
        _score_kernel,
        grid=(1,),
        in_specs=[
            pl.BlockSpec((_B, _C), lambda k: (0, 0)),
            pl.BlockSpec((_C, 16), lambda k: (0, 0)),
        ],
        out_specs=[
            pl.BlockSpec((_B, 16), lambda k: (0, 0)),
            pl.BlockSpec((_B, 1), lambda k: (0, 0)),
            pl.BlockSpec((_B, 1), lambda k: (0, 0)),
        ],
        out_shape=[
            jax.ShapeDtypeStruct((_B, 16), f32),
            jax.ShapeDtypeStruct((_B, 1), jnp.int32),
            jax.ShapeDtypeStruct((_B, 1), jnp.int32),
        ],
        interpret=interpret,
    )(gam, labx)

    return sc16[:, :_NCLS], pred.reshape(_B), clusters.reshape(_B)


def kernel(data, labels, mu, S, n, cluster_labels):
    del labels
    return _run(data, mu, S, n, cluster_labels)


# speedup vs baseline: 1.0171x; 1.0044x over previous
import functools

import jax
import jax.numpy as jnp
from jax import lax
from jax.experimental import pallas as pl
_D = 64
_NCLS = 10
_C = 1024
_KAPPA_N = 5.0
_S0 = 0.001
_B = 256
_CB = 128
_NBLK = _C // _CB


def _dist_kernel(sb_ref, data_ref, mu_ref, n_ref, gam_ref):
    data = data_ref[...]
    cols = []
    for ci in range(_CB):
        diff = data - mu_ref[ci:ci + 1, :]
        tmp = jnp.dot(diff.astype(jnp.bfloat16),
                      sb_ref[ci * _D:(ci + 1) * _D, :],
                      preferred_element_type=jnp.float32)
        cols.append(jnp.sum(tmp * diff, axis=1, keepdims=True))
    d2 = jnp.concatenate(cols, axis=1)
    match = jnp.where(n_ref[0] >= _KAPPA_N, 1.0, 0.0)
    gam_ref[...] = jnp.exp(-0.5 * d2) * match


def _score_kernel(gam_ref, lab_ref, sc_ref, pred_ref, cl_ref):
    g = gam_ref[...]
    den = jnp.sum(g, axis=1, keepdims=True) + 1e-12
    gn = (g / den).astype(jnp.bfloat16)
    sc = jnp.dot(gn, lab_ref[...], preferred_element_type=jnp.float32)
    sc_ref[...] = sc
    li = lax.broadcasted_iota(jnp.int32, (_B, 16), 1)
    scm = jnp.where(li < _NCLS, sc, -jnp.inf)
    m = jnp.max(scm, axis=1, keepdims=True)
    pred_ref[...] = jnp.min(jnp.where(scm == m, li, 2 ** 30),
                            axis=1, keepdims=True)
    ci = lax.broadcasted_iota(jnp.int32, (_B, _C), 1)
    gm = jnp.max(g, axis=1, keepdims=True)
    cl_ref[...] = jnp.min(jnp.where(g == gm, ci, 2 ** 30),
                          axis=1, keepdims=True)


@functools.partial(jax.jit, static_argnames=("interpret",))
def _run(data, mu, S, n, cluster_labels, interpret=False):
    f32 = jnp.float32
    n_c = n[:_C].astype(f32)
    Sigma = (S[:_C].astype(f32) / jnp.maximum(n_c, 1.0)[:, None, None]
             + _S0 * jnp.eye(_D, dtype=f32)[None])

    sb = jnp.linalg.inv(Sigma).astype(jnp.bfloat16).reshape(_C * _D, _D)

    n3 = n_c.reshape(_NBLK, 1, _CB)

    gam = pl.pallas_call(
        _dist_kernel,
        grid=(_NBLK,),
        in_specs=[
            pl.BlockSpec((_CB * _D, _D), lambda k: (k, 0)),
            pl.BlockSpec((_B, _D), lambda k: (0, 0)),
            pl.BlockSpec((_CB, _D), lambda k: (k, 0)),
            pl.BlockSpec((1, 1, _CB), lambda k: (k, 0, 0)),
        ],
        out_specs=pl.BlockSpec((_B, _CB), lambda k: (0, k)),
        out_shape=jax.ShapeDtypeStruct((_B, _C), f32),
        interpret=interpret,
    )(sb, data.astype(f32), mu[:_C].astype(f32), n3)

    lab = cluster_labels[:_C].astype(f32)
    labx = jnp.concatenate([lab, jnp.zeros((_C, 6), f32)],
                           axis=1).astype(jnp.bfloat16)

    sc16, pred, clusters = pl.pallas_call(
        _score_kernel,
        grid=(1,),
        in_specs=[
            pl.BlockSpec((_B, _C), lambda k: (0, 0)),
            pl.BlockSpec((_C, 16), lambda k: (0, 0)),
        ],
        out_specs=[
            pl.BlockSpec((_B, 16), lambda k: (0, 0)),
            pl.BlockSpec((_B, 1), lambda k: (0, 0)),
            pl.BlockSpec((_B, 1), lambda k: (0, 0)),
        ],
        out_shape=[
            jax.ShapeDtypeStruct((_B, 16), f32),
            jax.ShapeDtypeStruct((_B, 1), jnp.int32),
            jax.ShapeDtypeStruct((_B, 1), jnp.int32),
        ],
        interpret=interpret,
    )(gam, labx)

    return sc16[:, :_NCLS], pred.reshape(_B), clusters.reshape(_B)


def kernel(data, labels, mu, S, n, cluster_labels):
    del labels
    return _run(data, mu, S, n, cluster_labels)


# speedup vs baseline: 1.0172x; 1.0001x over previous
import functools

import jax
import jax.numpy as jnp
from jax import lax
from jax.experimental import pallas as pl
from jax.experimental.pallas import tpu as pltpu
_D = 64
_NCLS = 10
_C = 1024
_KAPPA_N = 5.0
_S0 = 0.001
_B = 256
_CB = 128
_NBLK = _C // _CB


def _fused_kernel(sb_ref, data_ref, mu_ref, n_ref, lab_ref,
                  sc_ref, pred_ref, cl_ref, gam_ref):
    k = pl.program_id(0)
    data = data_ref[...]
    cols = []
    for ci in range(_CB):
        diff = data - mu_ref[ci:ci + 1, :]
        tmp = jnp.dot(diff.astype(jnp.bfloat16),
                      sb_ref[ci * _D:(ci + 1) * _D, :],
                      preferred_element_type=jnp.float32)
        cols.append(jnp.sum(tmp * diff, axis=1, keepdims=True))
    d2 = jnp.concatenate(cols, axis=1)
    match = jnp.where(n_ref[0] >= _KAPPA_N, 1.0, 0.0)
    gam_ref[:, pl.ds(k * _CB, _CB)] = jnp.exp(-0.5 * d2) * match

    @pl.when(k == _NBLK - 1)
    def _score():
        g = gam_ref[...]
        den = jnp.sum(g, axis=1, keepdims=True) + 1e-12
        gn = (g / den).astype(jnp.bfloat16)
        sc = jnp.dot(gn, lab_ref[...], preferred_element_type=jnp.float32)
        sc_ref[...] = sc
        li = lax.broadcasted_iota(jnp.int32, (_B, 16), 1)
        scm = jnp.where(li < _NCLS, sc, -jnp.inf)
        m = jnp.max(scm, axis=1, keepdims=True)
        pred_ref[...] = jnp.min(jnp.where(scm == m, li, 2 ** 30),
                                axis=1, keepdims=True)
        ci = lax.broadcasted_iota(jnp.int32, (_B, _C), 1)
        gm = jnp.max(g, axis=1, keepdims=True)
        cl_ref[...] = jnp.min(jnp.where(g == gm, ci, 2 ** 30),
                              axis=1, keepdims=True)


@functools.partial(jax.jit, static_argnames=("interpret",))
def _run(data, mu, S, n, cluster_labels, interpret=False):
    f32 = jnp.float32
    n_c = n[:_C].astype(f32)
    Sigma = (S[:_C].astype(f32) / jnp.maximum(n_c, 1.0)[:, None, None]
             + _S0 * jnp.eye(_D, dtype=f32)[None])

    sb = jnp.linalg.inv(Sigma).astype(jnp.bfloat16).reshape(_C * _D, _D)

    n3 = n_c.reshape(_NBLK, 1, _CB)
    lab = cluster_labels[:_C].astype(f32)
    labx = jnp.concatenate([lab, jnp.zeros((_C, 6), f32)],
                           axis=1).astype(jnp.bfloat16)

    sc16, pred, clusters = pl.pallas_call(
        _fused_kernel,
        grid=(_NBLK,),
        in_specs=[
            pl.BlockSpec((_CB * _D, _D), lambda k: (k, 0)),
            pl.BlockSpec((_B, _D), lambda k: (0, 0)),
            pl.BlockSpec((_CB, _D), lambda k: (k, 0)),
            pl.BlockSpec((1, 1, _CB), lambda k: (k, 0, 0)),
            pl.BlockSpec((_C, 16), lambda k: (0, 0)),
        ],
        out_specs=[
            pl.BlockSpec((_B, 16), lambda k: (0, 0)),
            pl.BlockSpec((_B, 1), lambda k: (0, 0)),
            pl.BlockSpec((_B, 1), lambda k: (0, 0)),
        ],
        out_shape=[
            jax.ShapeDtypeStruct((_B, 16), f32),
            jax.ShapeDtypeStruct((_B, 1), jnp.int32),
            jax.ShapeDtypeStruct((_B, 1), jnp.int32),
        ],
        scratch_shapes=[pltpu.VMEM((_B, _C), f32)],
        interpret=interpret,
    )(sb, data.astype(f32), mu[:_C].astype(f32), n3, labx)

    return sc16[:, :_NCLS], pred.reshape(_B), clusters.reshape(_B)


def kernel(data, labels, mu, S, n, cluster_labels):
    del labels
    return _run(data, mu, S, n, cluster_labels)
